# baseline (speedup 1.0000x reference)
import dataclasses
import functools

import jax
import jax.numpy as jnp
from jax import lax
from jax.experimental import pallas as pl
from jax.experimental.pallas import tpu as pltpu
from jax.experimental.pallas import tpu_sc as plsc

NC = 2
NS = 16
LANES = 16


def _matmul(x, W):
    n, d_in = x.shape
    d_out = W.shape[1]
    blk = 2000

    def body(x_ref, w_ref, o_ref):
        o_ref[...] = jnp.dot(
            x_ref[...], w_ref[...],
            preferred_element_type=jnp.float32,
            precision=lax.Precision.HIGHEST,
        )

    return pl.pallas_call(
        body,
        grid=(n // blk,),
        in_specs=[
            pl.BlockSpec((blk, d_in), lambda i: (i, 0)),
            pl.BlockSpec((d_in, d_out), lambda i: (0, 0)),
        ],
        out_specs=pl.BlockSpec((blk, d_out), lambda i: (i, 0)),
        out_shape=jax.ShapeDtypeStruct((n, d_out), jnp.float32),
    )(x, W)


def _scatter_partials(xw, src, dst, ev, zeros):
    n, d = xw.shape
    e = src.shape[0]
    nw = NC * NS
    epw = e // nw
    chunk = 32
    nchunk = epw // chunk
    n_pad = zeros.shape[0]
    rows_per_sub = n_pad // NS
    nbuf = 8

    ngroup = 4
    g_e = epw // ngroup
    nchunk_g = g_e // chunk

    src = src.reshape(nw, ngroup, g_e)
    dst = dst.reshape(nw, ngroup, nchunk_g, chunk)
    ev = ev.reshape(nw, ngroup, g_e)

    mesh = plsc.VectorSubcoreMesh(core_axis_name="c", subcore_axis_name="s")
    cp = pltpu.CompilerParams()
    if "needs_layout_passes" in pltpu.CompilerParams.__dataclass_fields__:
        cp = dataclasses.replace(cp, needs_layout_passes=False)

    @functools.partial(
        pl.kernel,
        mesh=mesh,
        compiler_params=cp,
        out_type=jax.ShapeDtypeStruct((NC * n_pad, d), jnp.float32),
        scratch_types=[
            pltpu.VMEM((g_e,), jnp.int32),
            pltpu.VMEM((nchunk_g, chunk), jnp.int32),
            pltpu.VMEM((g_e,), jnp.float32),
            pltpu.VMEM((nbuf, chunk, d), jnp.float32),
            pltpu.VMEM_SHARED((n_pad, d), jnp.float32),
        ] + [pltpu.SemaphoreType.DMA] * (2 * nbuf),
    )
    def k(xw_hbm, src_hbm, dst_hbm, ev_hbm, z_hbm, out_hbm,
          src_v, dst_v, ev_v, rows_v, acc, *sems_all):
        c = lax.axis_index("c")
        s = lax.axis_index("s")
        my_rows = pl.ds(s * rows_per_sub, rows_per_sub)
        pltpu.sync_copy(z_hbm.at[my_rows], acc.at[my_rows])

        wid = s * NC + c
        plsc.subcore_barrier()

        gsems = list(sems_all[:nbuf])
        ssems = list(sems_all[nbuf:])

        def gather(ci, b):
            idx = src_v.at[pl.ds(ci * chunk, chunk)]
            return pltpu.make_async_copy(
                xw_hbm.at[idx], rows_v.at[b], gsems[b])

        def scatter(ci, b):
            return pltpu.make_async_copy(
                rows_v.at[b], acc.at[dst_v.at[ci]], ssems[b])

        def process(ci, b):
            gather(ci, b).wait()
            rows_b = rows_v.at[b]

            @pl.loop(0, chunk, step=4)
            def _(i):
                for t in range(4):
                    scale = plsc.load_gather(
                        ev_v,
                        [jnp.full((LANES,), ci * chunk + i + t, jnp.int32)])
                    for j in range(d // LANES):
                        sl = (i + t, pl.ds(j * LANES, LANES))
                        rows_b.at[*sl][...] = rows_b.at[*sl][...] * scale

            scatter(ci, b).start(add=True)
            pb = (b - 1) % nbuf

            @pl.when(ci >= 1)
            def _():
                scatter(ci - 1, pb).wait()

                @pl.when(ci + nbuf - 1 < nchunk_g)
                def _():
                    gather(ci + nbuf - 1, pb).start()

        @pl.loop(0, ngroup)
        def _(grp):
            pltpu.sync_copy(src_hbm.at[wid, grp], src_v)
            pltpu.sync_copy(dst_hbm.at[wid, grp], dst_v)
            pltpu.sync_copy(ev_hbm.at[wid, grp], ev_v)

            for b in range(nbuf):
                gather(b, b).start()

            @pl.loop(0, nchunk_g, step=nbuf)
            def _(g):
                for b in range(nbuf):
                    process(g + b, b)

            scatter(nchunk_g - 1, (nchunk_g - 1) % nbuf).wait()

        plsc.subcore_barrier()
        pltpu.sync_copy(
            acc.at[my_rows],
            out_hbm.at[pl.ds(c * n_pad + s * rows_per_sub, rows_per_sub)])

    return k(xw, src, dst, ev, zeros)


def _combine_relu(partials, n):
    d = partials.shape[-1]
    blk = 2000

    def body(p_ref, o_ref):
        o_ref[...] = jnp.maximum(p_ref[0] + p_ref[1], 0.0)

    return pl.pallas_call(
        body,
        grid=(n // blk,),
        in_specs=[pl.BlockSpec((NC, blk, d), lambda i: (0, i, 0))],
        out_specs=pl.BlockSpec((blk, d), lambda i: (i, 0)),
        out_shape=jax.ShapeDtypeStruct((n, d), jnp.float32),
    )(partials)


def kernel(x, edge_index, edge_values, W):
    n, _ = x.shape
    d = W.shape[1]
    xw = _matmul(x, W)
    n_pad = ((n + 8 * NS - 1) // (8 * NS)) * (8 * NS)
    e = edge_index.shape[1]
    nw = NC * NS
    epw = -(-e // (nw * 640)) * 640
    e_pad = nw * epw
    npad_rows = ((n + 8 * NS - 1) // (8 * NS)) * (8 * NS) - n
    fill = jnp.arange(e_pad - e, dtype=jnp.int32)
    src = jnp.concatenate([edge_index[1], fill % n])
    dst = jnp.concatenate([edge_index[0], n + fill % npad_rows])
    ev = jnp.concatenate(
        [edge_values, jnp.zeros((e_pad - e,), jnp.float32)])
    zeros = jnp.zeros((n_pad, d), jnp.float32)
    partials = _scatter_partials(xw, src, dst, ev, zeros)
    return _combine_relu(partials.reshape(NC, n_pad, d), n)


# speedup vs baseline: 1.0351x; 1.0351x over previous
import dataclasses
import functools

import jax
import jax.numpy as jnp
from jax import lax
from jax.experimental import pallas as pl
from jax.experimental.pallas import tpu as pltpu
from jax.experimental.pallas import tpu_sc as plsc

NC = 2
NS = 16
LANES = 16


def _matmul(x, W):
    n, d_in = x.shape
    d_out = W.shape[1]
    blk = 2000

    def body(x_ref, w_ref, o_ref):
        o_ref[...] = jnp.dot(
            x_ref[...], w_ref[...],
            preferred_element_type=jnp.float32,
            precision=lax.Precision.HIGHEST,
        )

    return pl.pallas_call(
        body,
        grid=(n // blk,),
        in_specs=[
            pl.BlockSpec((blk, d_in), lambda i: (i, 0)),
            pl.BlockSpec((d_in, d_out), lambda i: (0, 0)),
        ],
        out_specs=pl.BlockSpec((blk, d_out), lambda i: (i, 0)),
        out_shape=jax.ShapeDtypeStruct((n, d_out), jnp.float32),
    )(x, W)


def _scatter_partials(xw, src, dst, ev, zeros):
    n, d = xw.shape
    e = src.shape[0]
    nw = NC * NS
    epw = e // nw
    chunk = 64
    nchunk = epw // chunk
    n_pad = zeros.shape[0]
    rows_per_sub = n_pad // NS
    nbuf = 4

    ngroup = 4
    g_e = epw // ngroup
    nchunk_g = g_e // chunk

    src = src.reshape(nw, ngroup, g_e)
    dst = dst.reshape(nw, ngroup, nchunk_g, chunk)
    ev = ev.reshape(nw, ngroup, g_e)

    mesh = plsc.VectorSubcoreMesh(core_axis_name="c", subcore_axis_name="s")
    cp = pltpu.CompilerParams()
    if "needs_layout_passes" in pltpu.CompilerParams.__dataclass_fields__:
        cp = dataclasses.replace(cp, needs_layout_passes=False)

    @functools.partial(
        pl.kernel,
        mesh=mesh,
        compiler_params=cp,
        out_type=jax.ShapeDtypeStruct((NC * n_pad, d), jnp.float32),
        scratch_types=[
            pltpu.VMEM((g_e,), jnp.int32),
            pltpu.VMEM((nchunk_g, chunk), jnp.int32),
            pltpu.VMEM((g_e,), jnp.float32),
            pltpu.VMEM((nbuf, chunk, d), jnp.float32),
            pltpu.VMEM_SHARED((n_pad, d), jnp.float32),
        ] + [pltpu.SemaphoreType.DMA] * (2 * nbuf + 1),
    )
    def k(xw_hbm, src_hbm, dst_hbm, ev_hbm, z_hbm, out_hbm,
          src_v, dst_v, ev_v, rows_v, acc, *sems_all):
        c = lax.axis_index("c")
        s = lax.axis_index("s")
        my_rows = pl.ds(s * rows_per_sub, rows_per_sub)
        pltpu.sync_copy(z_hbm.at[my_rows], acc.at[my_rows])

        wid = s * NC + c
        plsc.subcore_barrier()

        gsems = list(sems_all[:nbuf])
        ssems = list(sems_all[nbuf:2 * nbuf])
        isem = sems_all[2 * nbuf]

        def gather(ci, b):
            idx = src_v.at[pl.ds(ci * chunk, chunk)]
            return pltpu.make_async_copy(
                xw_hbm.at[idx], rows_v.at[b], gsems[b])

        def scatter(ci, b):
            return pltpu.make_async_copy(
                rows_v.at[b], acc.at[dst_v.at[ci]], ssems[b])

        def process(ci, b):
            gather(ci, b).wait()
            rows_b = rows_v.at[b]

            @pl.loop(0, chunk, step=4)
            def _(i):
                for t in range(4):
                    scale = plsc.load_gather(
                        ev_v,
                        [jnp.full((LANES,), ci * chunk + i + t, jnp.int32)])
                    for j in range(d // LANES):
                        sl = (i + t, pl.ds(j * LANES, LANES))
                        rows_b.at[*sl][...] = rows_b.at[*sl][...] * scale

            scatter(ci, b).start(add=True)
            pb = (b - 1) % nbuf

            @pl.when(ci >= 1)
            def _():
                scatter(ci - 1, pb).wait()

                @pl.when(ci + 3 < nchunk_g)
                def _():
                    gather(ci + 3, pb).start()

        @pl.loop(0, ngroup)
        def _(grp):
            c1 = pltpu.make_async_copy(src_hbm.at[wid, grp], src_v, isem)
            c2 = pltpu.make_async_copy(dst_hbm.at[wid, grp], dst_v, isem)
            c3 = pltpu.make_async_copy(ev_hbm.at[wid, grp], ev_v, isem)
            c1.start(); c2.start(); c3.start()
            c1.wait(); c2.wait(); c3.wait()

            for b in range(nbuf):
                gather(b, b).start()

            @pl.loop(0, nchunk_g, step=nbuf)
            def _(g):
                for b in range(nbuf):
                    process(g + b, b)

            scatter(nchunk_g - 1, (nchunk_g - 1) % nbuf).wait()

        plsc.subcore_barrier()
        pltpu.sync_copy(
            acc.at[my_rows],
            out_hbm.at[pl.ds(c * n_pad + s * rows_per_sub, rows_per_sub)])

    return k(xw, src, dst, ev, zeros)


def _combine_relu(partials, n):
    d = partials.shape[-1]
    blk = 2000

    def body(p_ref, o_ref):
        o_ref[...] = jnp.maximum(p_ref[0] + p_ref[1], 0.0)

    return pl.pallas_call(
        body,
        grid=(n // blk,),
        in_specs=[pl.BlockSpec((NC, blk, d), lambda i: (0, i, 0))],
        out_specs=pl.BlockSpec((blk, d), lambda i: (i, 0)),
        out_shape=jax.ShapeDtypeStruct((n, d), jnp.float32),
    )(partials)


def kernel(x, edge_index, edge_values, W):
    n, _ = x.shape
    d = W.shape[1]
    xw = _matmul(x, W)
    n_pad = ((n + 8 * NS - 1) // (8 * NS)) * (8 * NS)
    e = edge_index.shape[1]
    nw = NC * NS
    epw = -(-e // (nw * 640)) * 640
    e_pad = nw * epw
    npad_rows = ((n + 8 * NS - 1) // (8 * NS)) * (8 * NS) - n
    fill = jnp.arange(e_pad - e, dtype=jnp.int32)
    src = jnp.concatenate([edge_index[1], fill % n])
    dst = jnp.concatenate([edge_index[0], n + fill % npad_rows])
    ev = jnp.concatenate(
        [edge_values, jnp.zeros((e_pad - e,), jnp.float32)])
    zeros = jnp.zeros((n_pad, d), jnp.float32)
    partials = _scatter_partials(xw, src, dst, ev, zeros)
    return _combine_relu(partials.reshape(NC, n_pad, d), n)
